# Initial kernel scaffold; baseline (speedup 1.0000x reference)
import jax
import jax.numpy as jnp
from jax.experimental import pallas as pl


def kernel(encode_h, yt, encode_len, W_tan, w_pt, W_ct2ht):
    raise NotImplementedError("write your pallas kernel here")


# speedup vs baseline: 1.2749x; 1.2749x over previous
import functools

import jax
import jax.numpy as jnp
from jax import lax
from jax.experimental import pallas as pl
from jax.experimental.pallas import tpu as pltpu
from jax.experimental.pallas import tpu_sc as plsc

B, S, H = 16, 4096, 1024
D = 64
W = 2 * D


def _k1_body(yt_ref, wtan_ref, wpt_ref, len_ref, idx_ref, bias_ref, gv_ref):
    yt = yt_ref[...]
    a = jnp.tanh(
        lax.dot_general(yt, wtan_ref[...], (((1,), (0,)), ((), ())),
                        preferred_element_type=jnp.float32))
    s = lax.dot_general(a, wpt_ref[...], (((1,), (0,)), ((), ())),
                        preferred_element_type=jnp.float32)
    lens_i = len_ref[...]
    pt = jax.nn.sigmoid(s) * lens_i.astype(jnp.float32)
    pti = jnp.floor(pt).astype(jnp.int32)
    left = jnp.maximum(0, pti - D)
    right = jnp.minimum(lens_i, pti + D)
    cols = lax.broadcasted_iota(jnp.int32, (B, W), 1)
    idx = left + cols
    valid = idx < right
    idx_c = jnp.clip(idx, 0, S - 1)
    rowbase = lax.broadcasted_iota(jnp.int32, (B, W), 0) * S
    idx_ref[...] = idx_c + rowbase
    bias_ref[...] = jnp.where(valid, 0.0, -1e30)
    gauss = jnp.exp(-((idx.astype(jnp.float32) - pt) ** 2) / (D * D / 2.0))
    gv_ref[...] = gauss * valid.astype(jnp.float32)


def _k3_body(g_ref, yt_ref, bias_ref, gv_ref, wct_ref, out_ref):
    yt = yt_ref[...]
    cts = []
    for b in range(B):
        g_b = g_ref[b * W:(b + 1) * W, :]
        ytb = yt[b:b + 1, :]
        s = lax.dot_general(ytb, g_b, (((1,), (1,)), ((), ())),
                            preferred_element_type=jnp.float32)
        s = s + bias_ref[b:b + 1, :]
        m = jnp.max(s, axis=1, keepdims=True)
        e = jnp.exp(s - m)
        z = jnp.sum(e, axis=1, keepdims=True)
        at = (e / z) * gv_ref[b:b + 1, :]
        ct = lax.dot_general(at, g_b, (((1,), (0,)), ((), ())),
                             preferred_element_type=jnp.float32)
        cts.append(ct)
    ct_all = jnp.concatenate(cts, axis=0)
    out_ref[...] = lax.dot_general(ct_all, wct_ref[...], (((1,), (0,)), ((), ())),
                                   preferred_element_type=jnp.float32)


def _make_sc_gather():
    info = plsc.get_sparse_core_info()
    nw = info.num_cores * info.num_subcores
    rows_total = B * W
    b_per_w = rows_total // nw
    mesh = plsc.VectorSubcoreMesh(core_axis_name="c", subcore_axis_name="s")

    @functools.partial(
        pl.kernel, mesh=mesh,
        out_type=jax.ShapeDtypeStruct((rows_total, H), jnp.float32),
        scratch_types=[
            pltpu.VMEM((b_per_w,), jnp.int32),
            pltpu.VMEM((b_per_w, H), jnp.float32),
            pltpu.SemaphoreType.DMA,
        ],
    )
    def gather_k(enc_hbm, idx_hbm, out_hbm, idx_v, rows_v, sem):
        wid = lax.axis_index("s") * info.num_cores + lax.axis_index("c")
        base = wid * b_per_w
        pltpu.sync_copy(idx_hbm.at[pl.ds(base, b_per_w)], idx_v)
        pltpu.async_copy(enc_hbm.at[idx_v], rows_v, sem).wait()
        pltpu.sync_copy(rows_v, out_hbm.at[pl.ds(base, b_per_w), :])

    return gather_k


def kernel(encode_h, yt, encode_len, W_tan, w_pt, W_ct2ht):
    enc2d = encode_h.reshape(B * S, H)
    lens2d = encode_len.reshape(B, 1)

    idx, bias, gv = pl.pallas_call(
        _k1_body,
        out_shape=[
            jax.ShapeDtypeStruct((B, W), jnp.int32),
            jax.ShapeDtypeStruct((B, W), jnp.float32),
            jax.ShapeDtypeStruct((B, W), jnp.float32),
        ],
    )(yt, W_tan, w_pt, lens2d)

    gathered = _make_sc_gather()(enc2d, idx.reshape(B * W))

    ht = pl.pallas_call(
        _k3_body,
        out_shape=jax.ShapeDtypeStruct((B, H), jnp.float32),
    )(gathered, yt, bias, gv, W_ct2ht)
    return ht
